# baseline (speedup 1.0000x reference)
import jax
import jax.numpy as jnp
from jax import lax
from jax.experimental import pallas as pl
from jax.experimental.pallas import tpu as pltpu
from jax.experimental.pallas import tpu_sc as plsc

N_NODES = 10000
N_EDGES = 320000
EMBED = 128
LANES = 16
CHUNK = 128
N_WORKERS = 32
CPW = 80
N_CHUNKS_PAD = CPW * N_WORKERS
PADE = N_CHUNKS_PAD * CHUNK
EPW = CPW * CHUNK
N_CHUNKS = N_EDGES // CHUNK
N_PAD = 10240
ROWS_PER_SUB = N_PAD // 16
SCALE = 1.0 / (EMBED ** 0.5)


def _proj_body(e_ref, wq_ref, wk_ref, wv_ref, q_ref, k_ref, v_ref):
    e = e_ref[...]
    dn = (((1,), (1,)), ((), ()))
    q_ref[...] = lax.dot_general(e, wq_ref[...], dn, preferred_element_type=jnp.float32)
    k_ref[...] = lax.dot_general(e, wk_ref[...], dn, preferred_element_type=jnp.float32)
    v_ref[...] = lax.dot_general(e, wv_ref[...], dn, preferred_element_type=jnp.float32)


def _project(e, wq, wk, wv):
    sds = jax.ShapeDtypeStruct((N_NODES, EMBED), jnp.float32)
    return pl.pallas_call(_proj_body, out_shape=[sds, sds, sds])(e, wq, wk, wv)


def _dot16(qrows, krows, e):
    acc = qrows[e, pl.ds(0, LANES)] * krows[e, pl.ds(0, LANES)]
    for j in range(1, EMBED // LANES):
        acc = acc + (qrows[e, pl.ds(j * LANES, LANES)]
                     * krows[e, pl.ds(j * LANES, LANES)])
    return acc


def _scores_body(q_hbm, k_hbm, src_hbm, dst_hbm, s_hbm,
                 src_v, dst_v, q0, q1, k0, k1, s_all,
                 sq0, sq1, sk0, sk1):
    c = lax.axis_index("c")
    s = lax.axis_index("s")
    wid = c * 16 + s
    lane = lax.iota(jnp.int32, LANES)

    pltpu.sync_copy(src_hbm.at[pl.ds(wid * CPW, CPW)], src_v)
    pltpu.sync_copy(dst_hbm.at[pl.ds(wid * CPW, CPW)], dst_v)

    qb, kb, sqb, skb = (q0, q1), (k0, k1), (sq0, sq1), (sk0, sk1)

    def issue(ch, b):
        cq = pltpu.async_copy(q_hbm.at[src_v.at[ch]], qb[b], sqb[b])
        ck = pltpu.async_copy(k_hbm.at[dst_v.at[ch]], kb[b], skb[b])
        return cq, ck

    def wait(ch, b):
        pltpu.make_async_copy(q_hbm.at[src_v.at[ch]], qb[b], sqb[b]).wait()
        pltpu.make_async_copy(k_hbm.at[dst_v.at[ch]], kb[b], skb[b]).wait()

    def compute(ch, b):
        qrows, krows = qb[b], kb[b]

        def group_body(g, _):
            grp = jnp.zeros((LANES,), jnp.float32)
            for l in range(LANES):
                acc = _dot16(qrows, krows, g * LANES + l)
                for st in (1, 2, 4, 8):
                    acc = acc + acc.at[lane ^ st].get(mode="promise_in_bounds")
                grp = jnp.where(lane == l, acc, grp)
            s_all[pl.ds(ch * CHUNK + g * LANES, LANES)] = grp * SCALE
            return 0

        lax.fori_loop(0, CHUNK // LANES, group_body, 0)

    issue(0, 0)

    def body(t, _):
        c0 = 2 * t
        issue(c0 + 1, 1)
        wait(c0, 0)
        compute(c0, 0)

        @pl.when(t < CPW // 2 - 1)
        def _():
            issue(c0 + 2, 0)

        wait(c0 + 1, 1)
        compute(c0 + 1, 1)
        return 0

    lax.fori_loop(0, CPW // 2, body, 0)
    pltpu.sync_copy(s_all, s_hbm.at[pl.ds(wid * EPW, EPW)])


def _edge_scores(q, k, src2, dst2):
    mesh = plsc.VectorSubcoreMesh(core_axis_name="c", subcore_axis_name="s")
    f = pl.kernel(
        _scores_body,
        out_type=jax.ShapeDtypeStruct((PADE,), jnp.float32),
        mesh=mesh,
        scratch_types=[
            pltpu.VMEM((CPW, CHUNK), jnp.int32),
            pltpu.VMEM((CPW, CHUNK), jnp.int32),
            pltpu.VMEM((CHUNK, EMBED), jnp.float32),
            pltpu.VMEM((CHUNK, EMBED), jnp.float32),
            pltpu.VMEM((CHUNK, EMBED), jnp.float32),
            pltpu.VMEM((CHUNK, EMBED), jnp.float32),
            pltpu.VMEM((EPW,), jnp.float32),
            pltpu.SemaphoreType.DMA,
            pltpu.SemaphoreType.DMA,
            pltpu.SemaphoreType.DMA,
            pltpu.SemaphoreType.DMA,
        ],
    )
    return f(q, k, src2, dst2)


def _softmax_body(s_ref, w_ref):
    sc = s_ref[...]
    m = jnp.max(sc)
    e = jnp.exp(sc - m)
    w_ref[...] = e / jnp.sum(e)


def _softmax(scores):
    s2 = scores.reshape(N_CHUNKS, CHUNK)
    w2 = pl.pallas_call(
        _softmax_body,
        out_shape=jax.ShapeDtypeStruct((N_CHUNKS, CHUNK), jnp.float32),
    )(s2)
    return w2.reshape(N_EDGES)


def _scatter_body(v_hbm, src_hbm, dst_hbm, w_hbm, out_hbm,
                  src_v, dr, wr, g0, g1, accum,
                  sg0, sg1, ss0, ss1):
    c = lax.axis_index("c")
    s = lax.axis_index("s")
    wid = c * 16 + s

    def zrow(r, _):
        for j in range(EMBED // LANES):
            g0[r, pl.ds(j * LANES, LANES)] = jnp.zeros((LANES,), jnp.float32)
        return 0

    lax.fori_loop(0, CHUNK, zrow, 0)
    for kk in range(ROWS_PER_SUB // CHUNK):
        pltpu.sync_copy(g0, accum.at[pl.ds(s * ROWS_PER_SUB + kk * CHUNK, CHUNK)])
    plsc.subcore_barrier()

    pltpu.sync_copy(src_hbm.at[pl.ds(wid * CPW, CPW)], src_v)

    gb, sgb, srb = (g0, g1), (sg0, sg1), (ss0, ss1)

    def prep(ch, b):
        pltpu.sync_copy(dst_hbm.at[pl.ds(wid * CPW + ch, 1)], dr.at[pl.ds(b, 1)])
        pltpu.sync_copy(w_hbm.at[pl.ds(wid * CPW + ch, 1)], wr.at[pl.ds(b, 1)])
        pltpu.async_copy(v_hbm.at[dr.at[b]], gb[b], sgb[b])

    def wait_g(b):
        pltpu.make_async_copy(v_hbm.at[dr.at[b]], gb[b], sgb[b]).wait()

    def issue_s(ch, b):
        pltpu.async_copy(gb[b], accum.at[src_v.at[ch]], srb[b], add=True)

    def wait_s(ch, b):
        pltpu.make_async_copy(gb[b], accum.at[src_v.at[ch]], srb[b]).wait()

    def scale(b):
        rows = gb[b]

        def group_body(g, _):
            wvec = wr[b, pl.ds(g * LANES, LANES)]
            for l in range(LANES):
                e = g * LANES + l
                we = wvec[l]
                for j in range(EMBED // LANES):
                    sl = pl.ds(j * LANES, LANES)
                    rows[e, sl] = rows[e, sl] * we
            return 0

        lax.fori_loop(0, CHUNK // LANES, group_body, 0)

    prep(0, 0)

    def body(t, _):
        c0 = 2 * t

        @pl.when(t > 0)
        def _():
            wait_s(c0 - 1, 1)

        prep(c0 + 1, 1)
        wait_g(0)
        scale(0)
        issue_s(c0, 0)

        @pl.when(t < CPW // 2 - 1)
        def _():
            wait_s(c0, 0)
            prep(c0 + 2, 0)

        wait_g(1)
        scale(1)
        issue_s(c0 + 1, 1)
        return 0

    lax.fori_loop(0, CPW // 2, body, 0)
    wait_s(CPW - 2, 0)
    wait_s(CPW - 1, 1)
    plsc.subcore_barrier()
    pltpu.sync_copy(accum.at[pl.ds(s * ROWS_PER_SUB, ROWS_PER_SUB)],
                    out_hbm.at[c, pl.ds(s * ROWS_PER_SUB, ROWS_PER_SUB)])


def _scatter(v, src2, dst2, w2):
    mesh = plsc.VectorSubcoreMesh(core_axis_name="c", subcore_axis_name="s")
    f = pl.kernel(
        _scatter_body,
        out_type=jax.ShapeDtypeStruct((2, N_PAD, EMBED), jnp.float32),
        mesh=mesh,
        scratch_types=[
            pltpu.VMEM((CPW, CHUNK), jnp.int32),
            pltpu.VMEM((2, CHUNK), jnp.int32),
            pltpu.VMEM((2, CHUNK), jnp.float32),
            pltpu.VMEM((CHUNK, EMBED), jnp.float32),
            pltpu.VMEM((CHUNK, EMBED), jnp.float32),
            pltpu.VMEM_SHARED((N_PAD, EMBED), jnp.float32),
            pltpu.SemaphoreType.DMA,
            pltpu.SemaphoreType.DMA,
            pltpu.SemaphoreType.DMA,
            pltpu.SemaphoreType.DMA,
        ],
    )
    return f(v, src2, dst2, w2)


def _combine_body(p_ref, o_ref):
    o_ref[...] = p_ref[0, :N_NODES] + p_ref[1, :N_NODES]


def _combine(parts):
    return pl.pallas_call(
        _combine_body,
        out_shape=jax.ShapeDtypeStruct((N_NODES, EMBED), jnp.float32),
    )(parts)


@jax.jit
def kernel(embeddings, edge_index, Wq, Wk, Wv):
    zpad = jnp.zeros((PADE - N_EDGES,), jnp.int32)
    src2 = jnp.concatenate([edge_index[0].astype(jnp.int32), zpad]
                           ).reshape(N_CHUNKS_PAD, CHUNK)
    dst2 = jnp.concatenate([edge_index[1].astype(jnp.int32), zpad]
                           ).reshape(N_CHUNKS_PAD, CHUNK)
    q, k, v = _project(embeddings, Wq, Wk, Wv)
    scores = _edge_scores(q, k, src2, dst2)
    w = _softmax(scores[:N_EDGES])
    w2 = jnp.concatenate([w, jnp.zeros((PADE - N_EDGES,), jnp.float32)]
                         ).reshape(N_CHUNKS_PAD, CHUNK)
    parts = _scatter(v, src2, dst2, w2)
    return _combine(parts)


# speedup vs baseline: 1.1100x; 1.1100x over previous
import jax
import jax.numpy as jnp
from jax import lax
from jax.experimental import pallas as pl
from jax.experimental.pallas import tpu as pltpu
from jax.experimental.pallas import tpu_sc as plsc

N_NODES = 10000
N_EDGES = 320000
EMBED = 128
LANES = 16
CHUNK = 128
N_WORKERS = 32
CPW_F = 104
CPW_S = 56
FAST_TOT = 16 * CPW_F
N_CHUNKS_PAD = FAST_TOT + 16 * CPW_S
PADE = N_CHUNKS_PAD * CHUNK
N_CHUNKS = N_EDGES // CHUNK
N_PAD = 10240
ROWS_PER_SUB = N_PAD // 16
SCALE = 1.0 / (EMBED ** 0.5)


def _proj_body(e_ref, wq_ref, wk_ref, wv_ref, q_ref, k_ref, v_ref):
    e = e_ref[...]
    dn = (((1,), (1,)), ((), ()))
    q_ref[...] = lax.dot_general(e, wq_ref[...], dn, preferred_element_type=jnp.float32)
    k_ref[...] = lax.dot_general(e, wk_ref[...], dn, preferred_element_type=jnp.float32)
    v_ref[...] = lax.dot_general(e, wv_ref[...], dn, preferred_element_type=jnp.float32)


def _project(e, wq, wk, wv):
    sds = jax.ShapeDtypeStruct((N_NODES, EMBED), jnp.float32)
    return pl.pallas_call(_proj_body, out_shape=[sds, sds, sds])(e, wq, wk, wv)


def _dot16(qrows, krows, e):
    acc = qrows[e, pl.ds(0, LANES)] * krows[e, pl.ds(0, LANES)]
    for j in range(1, EMBED // LANES):
        acc = acc + (qrows[e, pl.ds(j * LANES, LANES)]
                     * krows[e, pl.ds(j * LANES, LANES)])
    return acc


def _scores_body(q_hbm, k_hbm, src_hbm, dst_hbm, s_hbm,
                 src_v, dst_v, q0, q1, k0, k1, s_all,
                 sq0, sq1, sk0, sk1):
    c = lax.axis_index("c")
    s = lax.axis_index("s")
    base = jnp.where(c == 0, s * CPW_F, FAST_TOT + s * CPW_S)
    cnt = jnp.where(c == 0, CPW_F, CPW_S)
    lane = lax.iota(jnp.int32, LANES)

    @pl.when(c == 0)
    def _():
        pltpu.sync_copy(src_hbm.at[pl.ds(base, CPW_F)], src_v)
        pltpu.sync_copy(dst_hbm.at[pl.ds(base, CPW_F)], dst_v)

    @pl.when(c != 0)
    def _():
        pltpu.sync_copy(src_hbm.at[pl.ds(base, CPW_S)], src_v.at[pl.ds(0, CPW_S)])
        pltpu.sync_copy(dst_hbm.at[pl.ds(base, CPW_S)], dst_v.at[pl.ds(0, CPW_S)])

    qb, kb, sqb, skb = (q0, q1), (k0, k1), (sq0, sq1), (sk0, sk1)

    def issue(ch, b):
        cq = pltpu.async_copy(q_hbm.at[src_v.at[ch]], qb[b], sqb[b])
        ck = pltpu.async_copy(k_hbm.at[dst_v.at[ch]], kb[b], skb[b])
        return cq, ck

    def wait(ch, b):
        pltpu.make_async_copy(q_hbm.at[src_v.at[ch]], qb[b], sqb[b]).wait()
        pltpu.make_async_copy(k_hbm.at[dst_v.at[ch]], kb[b], skb[b]).wait()

    def compute(ch, b):
        qrows, krows = qb[b], kb[b]

        def group_body(g, _):
            grp = jnp.zeros((LANES,), jnp.float32)
            for l in range(LANES):
                acc = _dot16(qrows, krows, g * LANES + l)
                for st in (1, 2, 4, 8):
                    acc = acc + acc.at[lane ^ st].get(mode="promise_in_bounds")
                grp = jnp.where(lane == l, acc, grp)
            s_all[pl.ds(ch * CHUNK + g * LANES, LANES)] = grp * SCALE
            return 0

        lax.fori_loop(0, CHUNK // LANES, group_body, 0)

    issue(0, 0)

    def body(t, _):
        c0 = 2 * t

        @pl.when(c0 < cnt)
        def _():
            issue(c0 + 1, 1)
            wait(c0, 0)
            compute(c0, 0)

            @pl.when(c0 + 2 < cnt)
            def _():
                issue(c0 + 2, 0)

            wait(c0 + 1, 1)
            compute(c0 + 1, 1)

        return 0

    lax.fori_loop(0, CPW_F // 2, body, 0)

    @pl.when(c == 0)
    def _():
        pltpu.sync_copy(s_all, s_hbm.at[pl.ds(base * CHUNK, CPW_F * CHUNK)])

    @pl.when(c != 0)
    def _():
        pltpu.sync_copy(s_all.at[pl.ds(0, CPW_S * CHUNK)],
                        s_hbm.at[pl.ds(base * CHUNK, CPW_S * CHUNK)])


def _edge_scores(q, k, src2, dst2):
    mesh = plsc.VectorSubcoreMesh(core_axis_name="c", subcore_axis_name="s")
    f = pl.kernel(
        _scores_body,
        out_type=jax.ShapeDtypeStruct((PADE,), jnp.float32),
        mesh=mesh,
        scratch_types=[
            pltpu.VMEM((CPW_F, CHUNK), jnp.int32),
            pltpu.VMEM((CPW_F, CHUNK), jnp.int32),
            pltpu.VMEM((CHUNK, EMBED), jnp.float32),
            pltpu.VMEM((CHUNK, EMBED), jnp.float32),
            pltpu.VMEM((CHUNK, EMBED), jnp.float32),
            pltpu.VMEM((CHUNK, EMBED), jnp.float32),
            pltpu.VMEM((CPW_F * CHUNK,), jnp.float32),
            pltpu.SemaphoreType.DMA,
            pltpu.SemaphoreType.DMA,
            pltpu.SemaphoreType.DMA,
            pltpu.SemaphoreType.DMA,
        ],
    )
    return f(q, k, src2, dst2)


def _softmax_body(s_ref, w_ref):
    sc = s_ref[...]
    m = jnp.max(sc)
    e = jnp.exp(sc - m)
    w_ref[...] = e / jnp.sum(e)


def _softmax(scores):
    s2 = scores.reshape(N_CHUNKS, CHUNK)
    w2 = pl.pallas_call(
        _softmax_body,
        out_shape=jax.ShapeDtypeStruct((N_CHUNKS, CHUNK), jnp.float32),
    )(s2)
    return w2.reshape(N_EDGES)


def _scatter_body(v_hbm, src_hbm, dst_hbm, w_hbm, out_hbm,
                  src_v, dr, wr, g0, g1, accum,
                  sg0, sg1, ss0, ss1):
    c = lax.axis_index("c")
    s = lax.axis_index("s")
    base = jnp.where(c == 0, s * CPW_F, FAST_TOT + s * CPW_S)
    cnt = jnp.where(c == 0, CPW_F, CPW_S)

    def zrow(r, _):
        for j in range(EMBED // LANES):
            g0[r, pl.ds(j * LANES, LANES)] = jnp.zeros((LANES,), jnp.float32)
        return 0

    lax.fori_loop(0, CHUNK, zrow, 0)
    for kk in range(ROWS_PER_SUB // CHUNK):
        pltpu.sync_copy(g0, accum.at[pl.ds(s * ROWS_PER_SUB + kk * CHUNK, CHUNK)])
    plsc.subcore_barrier()

    @pl.when(c == 0)
    def _():
        pltpu.sync_copy(src_hbm.at[pl.ds(base, CPW_F)], src_v)

    @pl.when(c != 0)
    def _():
        pltpu.sync_copy(src_hbm.at[pl.ds(base, CPW_S)], src_v.at[pl.ds(0, CPW_S)])

    gb, sgb, srb = (g0, g1), (sg0, sg1), (ss0, ss1)

    def prep(ch, b):
        pltpu.sync_copy(dst_hbm.at[pl.ds(base + ch, 1)], dr.at[pl.ds(b, 1)])
        pltpu.sync_copy(w_hbm.at[pl.ds(base + ch, 1)], wr.at[pl.ds(b, 1)])
        pltpu.async_copy(v_hbm.at[dr.at[b]], gb[b], sgb[b])

    def wait_g(b):
        pltpu.make_async_copy(v_hbm.at[dr.at[b]], gb[b], sgb[b]).wait()

    def issue_s(ch, b):
        pltpu.async_copy(gb[b], accum.at[src_v.at[ch]], srb[b], add=True)

    def wait_s(ch, b):
        pltpu.make_async_copy(gb[b], accum.at[src_v.at[ch]], srb[b]).wait()

    def scale(b):
        rows = gb[b]

        def group_body(g, _):
            wvec = wr[b, pl.ds(g * LANES, LANES)]
            for l in range(LANES):
                e = g * LANES + l
                we = wvec[l]
                for j in range(EMBED // LANES):
                    sl = pl.ds(j * LANES, LANES)
                    rows[e, sl] = rows[e, sl] * we
            return 0

        lax.fori_loop(0, CHUNK // LANES, group_body, 0)

    prep(0, 0)

    def body(t, _):
        c0 = 2 * t

        @pl.when(c0 < cnt)
        def _():
            @pl.when(c0 > 0)
            def _():
                wait_s(c0 - 1, 1)

            prep(c0 + 1, 1)
            wait_g(0)
            scale(0)
            issue_s(c0, 0)

            @pl.when(c0 + 2 < cnt)
            def _():
                wait_s(c0, 0)
                prep(c0 + 2, 0)

            wait_g(1)
            scale(1)
            issue_s(c0 + 1, 1)

        return 0

    lax.fori_loop(0, CPW_F // 2, body, 0)
    wait_s(cnt - 2, 0)
    wait_s(cnt - 1, 1)
    plsc.subcore_barrier()
    pltpu.sync_copy(accum.at[pl.ds(s * ROWS_PER_SUB, ROWS_PER_SUB)],
                    out_hbm.at[c, pl.ds(s * ROWS_PER_SUB, ROWS_PER_SUB)])


def _scatter(v, src2, dst2, w2):
    mesh = plsc.VectorSubcoreMesh(core_axis_name="c", subcore_axis_name="s")
    f = pl.kernel(
        _scatter_body,
        out_type=jax.ShapeDtypeStruct((2, N_PAD, EMBED), jnp.float32),
        mesh=mesh,
        scratch_types=[
            pltpu.VMEM((CPW_F, CHUNK), jnp.int32),
            pltpu.VMEM((2, CHUNK), jnp.int32),
            pltpu.VMEM((2, CHUNK), jnp.float32),
            pltpu.VMEM((CHUNK, EMBED), jnp.float32),
            pltpu.VMEM((CHUNK, EMBED), jnp.float32),
            pltpu.VMEM_SHARED((N_PAD, EMBED), jnp.float32),
            pltpu.SemaphoreType.DMA,
            pltpu.SemaphoreType.DMA,
            pltpu.SemaphoreType.DMA,
            pltpu.SemaphoreType.DMA,
        ],
    )
    return f(v, src2, dst2, w2)


def _combine_body(p_ref, o_ref):
    o_ref[...] = p_ref[0, :N_NODES] + p_ref[1, :N_NODES]


def _combine(parts):
    return pl.pallas_call(
        _combine_body,
        out_shape=jax.ShapeDtypeStruct((N_NODES, EMBED), jnp.float32),
    )(parts)


@jax.jit
def kernel(embeddings, edge_index, Wq, Wk, Wv):
    zpad = jnp.zeros((PADE - N_EDGES,), jnp.int32)
    src2 = jnp.concatenate([edge_index[0].astype(jnp.int32), zpad]
                           ).reshape(N_CHUNKS_PAD, CHUNK)
    dst2 = jnp.concatenate([edge_index[1].astype(jnp.int32), zpad]
                           ).reshape(N_CHUNKS_PAD, CHUNK)
    q, k, v = _project(embeddings, Wq, Wk, Wv)
    scores = _edge_scores(q, k, src2, dst2)
    w = _softmax(scores[:N_EDGES])
    w2 = jnp.concatenate([w, jnp.zeros((PADE - N_EDGES,), jnp.float32)]
                         ).reshape(N_CHUNKS_PAD, CHUNK)
    parts = _scatter(v, src2, dst2, w2)
    return _combine(parts)
